# baseline (speedup 1.0000x reference)
import jax
import jax.numpy as jnp
from jax import lax
from jax.experimental import pallas as pl
from jax.experimental.pallas import tpu as pltpu
from jax.experimental.pallas import tpu_sc as plsc

VOCAB = 1000
D_MODEL = 64
SEQ = 50
NUM_CORES = 2
NUM_SUBCORES = 16
NUM_WORKERS = NUM_CORES * NUM_SUBCORES
ROWS_PER_W = 1024 // NUM_WORKERS
NSLOT = 8
AHEAD = 4


GSZ = 4
GROUPS = ROWS_PER_W // GSZ
RING = 4


def _emb_body(idx_hbm, table_hbm, out_hbm, idx_v, slots, gsem, osem):
    wid = lax.axis_index("s") * NUM_CORES + lax.axis_index("c")
    base = wid * ROWS_PER_W
    pltpu.sync_copy(idx_hbm.at[pl.ds(base, ROWS_PER_W)], idx_v)

    def gather(j):
        return pltpu.async_copy(
            table_hbm.at[idx_v.at[j]],
            slots.at[pl.ds((j % (RING * GSZ)) * SEQ, SEQ)],
            gsem,
        )

    def put(g):
        return pltpu.async_copy(
            slots.at[pl.ds((g % RING) * GSZ * SEQ, GSZ * SEQ)],
            out_hbm.at[pl.ds((base + g * GSZ) * SEQ, GSZ * SEQ)],
            osem,
        )

    gets, puts = {}, {}
    waited = set()
    for g in range(RING - 1):
        for r in range(GSZ):
            gets[g * GSZ + r] = gather(g * GSZ + r)
    for g in range(GROUPS):
        for r in range(GSZ):
            gets[g * GSZ + r].wait()
        puts[g] = put(g)
        ng = g + RING - 1
        if ng < GROUPS:
            pg = ng - RING
            if pg >= 0:
                puts[pg].wait()
                waited.add(pg)
            for r in range(GSZ):
                gets[ng * GSZ + r] = gather(ng * GSZ + r)
    for g in range(GROUPS):
        if g not in waited:
            puts[g].wait()


@jax.jit
def kernel(token_ids, w):
    nxt = jnp.concatenate([w[1:], jnp.zeros((1, D_MODEL), w.dtype)], axis=0)
    table2 = jnp.concatenate([w, nxt], axis=1)
    grab = pl.kernel(
        _emb_body,
        out_type=jax.ShapeDtypeStruct((1024 * SEQ, 2 * D_MODEL), jnp.float32),
        mesh=plsc.VectorSubcoreMesh(
            core_axis_name="c",
            subcore_axis_name="s",
            num_cores=NUM_CORES,
            num_subcores=NUM_SUBCORES,
        ),
        scratch_types=[
            pltpu.VMEM((ROWS_PER_W, SEQ), jnp.int32),
            pltpu.VMEM((RING * GSZ * SEQ, 2 * D_MODEL), jnp.float32),
            pltpu.SemaphoreType.DMA,
            pltpu.SemaphoreType.DMA,
        ],
        compiler_params=pltpu.CompilerParams(use_tc_tiling_on_sc=True),
    )
    out2 = grab(token_ids, table2)
    return out2[:, :D_MODEL].reshape(1024, SEQ, D_MODEL)


# speedup vs baseline: 1.3020x; 1.3020x over previous
import jax
import jax.numpy as jnp
from jax import lax
from jax.experimental import pallas as pl
from jax.experimental.pallas import tpu as pltpu
from jax.experimental.pallas import tpu_sc as plsc

VOCAB = 1000
D_MODEL = 64
SEQ = 50
NUM_CORES = 2
NUM_SUBCORES = 16
NUM_WORKERS = NUM_CORES * NUM_SUBCORES
ROWS_PER_W = 1024 // NUM_WORKERS


def _emb_body(idx_hbm, table_hbm, out_hbm, idx_v, rows_v, sem):
    wid = lax.axis_index("s") * NUM_CORES + lax.axis_index("c")
    base = wid * ROWS_PER_W
    pltpu.sync_copy(idx_hbm.at[pl.ds(base, ROWS_PER_W)], idx_v)
    copies = []
    for j in range(ROWS_PER_W):
        copies.append(
            pltpu.async_copy(table_hbm.at[idx_v.at[j]], rows_v.at[j], sem)
        )
    for c in copies:
        c.wait()
    pltpu.sync_copy(rows_v, out_hbm.at[pl.ds(base, ROWS_PER_W)])


@jax.jit
def kernel(token_ids, w):
    grab = pl.kernel(
        _emb_body,
        out_type=jax.ShapeDtypeStruct((1024, SEQ, D_MODEL), jnp.float32),
        mesh=plsc.VectorSubcoreMesh(
            core_axis_name="c",
            subcore_axis_name="s",
            num_cores=NUM_CORES,
            num_subcores=NUM_SUBCORES,
        ),
        scratch_types=[
            pltpu.VMEM((ROWS_PER_W, SEQ), jnp.int32),
            pltpu.VMEM((ROWS_PER_W, SEQ, D_MODEL), jnp.float32),
            pltpu.SemaphoreType.DMA,
        ],
        compiler_params=pltpu.CompilerParams(
            use_tc_tiling_on_sc=False,
            needs_layout_passes=False,
        ),
    )
    return grab(token_ids, w)
